# baseline (speedup 1.0000x reference)
import functools

import jax
import jax.numpy as jnp
from jax import lax
from jax.experimental import pallas as pl
from jax.experimental.pallas import tpu as pltpu
from jax.experimental.pallas import tpu_sc as plsc

_LANES = 16


def _sc_call(x, w):
    B, dim_x = x.shape
    V = w.shape[1]
    G = dim_x // 2
    info = plsc.get_sparse_core_info()
    NC, NS = info.num_cores, info.num_subcores
    NW = NC * NS
    rows_per_w = B // NW
    R = 2
    n_chunks = rows_per_w // R
    n_vec = V // _LANES

    mesh = plsc.VectorSubcoreMesh(core_axis_name="c", subcore_axis_name="s")

    @functools.partial(
        pl.kernel,
        mesh=mesh,
        out_type=jax.ShapeDtypeStruct((B * G * V,), jnp.float32),
        scratch_types=[
            pltpu.VMEM((dim_x, V), jnp.float32),
            pltpu.VMEM((rows_per_w, dim_x), jnp.float32),
            pltpu.VMEM((R * G * V,), jnp.float32),
            pltpu.VMEM((R * G * V,), jnp.float32),
            pltpu.SemaphoreType.DMA,
            pltpu.SemaphoreType.DMA,
        ],
    )
    def k(x_hbm, w_hbm, out_hbm, w_v, x_v, o_v0, o_v1, sem0, sem1):
        c = lax.axis_index("c")
        s = lax.axis_index("s")
        wid = s * NC + c
        base = wid * rows_per_w
        pltpu.sync_copy(w_hbm, w_v)
        pltpu.sync_copy(x_hbm.at[pl.ds(base, rows_per_w)], x_v)
        o_bufs = (o_v0, o_v1)
        sems = (sem0, sem1)

        def compute_chunk(ci, o_v):
            @plsc.parallel_loop(0, G // 8, 1, unroll=2)
            def gg_body(gg):
                xvs = [x_v[ci * R + r,
                           pl.ds(pl.multiple_of(gg * _LANES, _LANES), _LANES)]
                       for r in range(R)]
                for u in range(8):
                    g = gg * 8 + u
                    we = [w_v[2 * g, pl.ds(j * _LANES, _LANES)] for j in range(n_vec)]
                    wo = [w_v[2 * g + 1, pl.ds(j * _LANES, _LANES)] for j in range(n_vec)]
                    for r in range(R):
                        xe = xvs[r][2 * u]
                        xo = xvs[r][2 * u + 1]
                        for j in range(n_vec):
                            off = pl.multiple_of(
                                (r * G + g) * V + j * _LANES, _LANES)
                            o_v[pl.ds(off, _LANES)] = xe * we[j] + xo * wo[j]

        def pair_body(p, carry):
            for half in range(2):
                ci = 2 * p + half
                row0 = base + ci * R
                dma = pltpu.make_async_copy(
                    o_bufs[half], out_hbm.at[pl.ds(row0 * G * V, R * G * V)],
                    sems[half])

                @pl.when(p > 0)
                def _wait_prev():
                    dma.wait()

                compute_chunk(ci, o_bufs[half])
                dma.start()
            return carry

        lax.fori_loop(0, n_chunks // 2, pair_body, 0)
        last = (base + (n_chunks - 2) * R) * G * V
        pltpu.make_async_copy(
            o_bufs[0], out_hbm.at[pl.ds(last, R * G * V)], sems[0]).wait()
        pltpu.make_async_copy(
            o_bufs[1], out_hbm.at[pl.ds(last + R * G * V, R * G * V)],
            sems[1]).wait()

    return k(x, w)


def _tc_body(xe_ref, xo_ref, we_ref, wo_ref, o_ref):
    xe = xe_ref[...]
    xo = xo_ref[...]
    we = we_ref[...]
    wo = wo_ref[...]
    o_ref[...] = xe[:, :, None] * we[None, :, :] + xo[:, :, None] * wo[None, :, :]


def _tc_call(x, w):
    B, dim_x = x.shape
    V = w.shape[1]
    G = dim_x // 2
    BB = 256
    xr = x.reshape(B, G, 2)
    xe, xo = xr[:, :, 0], xr[:, :, 1]
    wr = w.reshape(G, 2, V)
    we, wo = wr[:, 0, :], wr[:, 1, :]
    return pl.pallas_call(
        _tc_body,
        grid=(B // BB,),
        in_specs=[
            pl.BlockSpec((BB, G), lambda i: (i, 0)),
            pl.BlockSpec((BB, G), lambda i: (i, 0)),
            pl.BlockSpec((G, V), lambda i: (0, 0)),
            pl.BlockSpec((G, V), lambda i: (0, 0)),
        ],
        out_specs=pl.BlockSpec((BB, G, V), lambda i: (i, 0, 0)),
        out_shape=jax.ShapeDtypeStruct((B, G, V), jnp.float32),
    )(xe, xo, we, wo)


def kernel(x, w, ind):
    B, dim_x = x.shape
    V = w.shape[1]
    G = dim_x // 2
    S = B // 2
    sc_part = _sc_call(x[:S], w).reshape(S, G, V)
    tc_part = _tc_call(x[S:], w)
    return jnp.concatenate([sc_part, tc_part], axis=0)


# speedup vs baseline: 1.9278x; 1.9278x over previous
import functools

import jax
import jax.numpy as jnp
from jax import lax
from jax.experimental import pallas as pl
from jax.experimental.pallas import tpu as pltpu
from jax.experimental.pallas import tpu_sc as plsc

_LANES = 16


def _sc_call(x, w):
    B, dim_x = x.shape
    V = w.shape[1]
    G = dim_x // 2
    info = plsc.get_sparse_core_info()
    NC, NS = info.num_cores, info.num_subcores
    NW = NC * NS
    rows_per_w = B // NW
    R = 2
    n_chunks = rows_per_w // R
    n_vec = V // _LANES

    mesh = plsc.VectorSubcoreMesh(core_axis_name="c", subcore_axis_name="s")

    @functools.partial(
        pl.kernel,
        mesh=mesh,
        out_type=jax.ShapeDtypeStruct((B * G * V,), jnp.float32),
        scratch_types=[
            pltpu.VMEM((dim_x, V), jnp.float32),
            pltpu.VMEM((rows_per_w, dim_x), jnp.float32),
            pltpu.VMEM((R * G * V,), jnp.float32),
            pltpu.VMEM((R * G * V,), jnp.float32),
            pltpu.SemaphoreType.DMA,
            pltpu.SemaphoreType.DMA,
        ],
    )
    def k(x_hbm, w_hbm, out_hbm, w_v, x_v, o_v0, o_v1, sem0, sem1):
        c = lax.axis_index("c")
        s = lax.axis_index("s")
        wid = s * NC + c
        base = wid * rows_per_w
        pltpu.sync_copy(w_hbm, w_v)
        pltpu.sync_copy(x_hbm.at[pl.ds(base, rows_per_w)], x_v)
        o_bufs = (o_v0, o_v1)
        sems = (sem0, sem1)

        def compute_chunk(ci, o_v):
            @plsc.parallel_loop(0, G // 8, 1, unroll=2)
            def gg_body(gg):
                xvs = [x_v[ci * R + r,
                           pl.ds(pl.multiple_of(gg * _LANES, _LANES), _LANES)]
                       for r in range(R)]
                for u in range(8):
                    g = gg * 8 + u
                    we = [w_v[2 * g, pl.ds(j * _LANES, _LANES)] for j in range(n_vec)]
                    wo = [w_v[2 * g + 1, pl.ds(j * _LANES, _LANES)] for j in range(n_vec)]
                    for r in range(R):
                        xe = xvs[r][2 * u]
                        xo = xvs[r][2 * u + 1]
                        for j in range(n_vec):
                            off = pl.multiple_of(
                                (r * G + g) * V + j * _LANES, _LANES)
                            o_v[pl.ds(off, _LANES)] = xe * we[j] + xo * wo[j]

        def pair_body(p, carry):
            for half in range(2):
                ci = 2 * p + half
                row0 = base + ci * R
                dma = pltpu.make_async_copy(
                    o_bufs[half], out_hbm.at[pl.ds(row0 * G * V, R * G * V)],
                    sems[half])

                @pl.when(p > 0)
                def _wait_prev():
                    dma.wait()

                compute_chunk(ci, o_bufs[half])
                dma.start()
            return carry

        lax.fori_loop(0, n_chunks // 2, pair_body, 0)
        last = (base + (n_chunks - 2) * R) * G * V
        pltpu.make_async_copy(
            o_bufs[0], out_hbm.at[pl.ds(last, R * G * V)], sems[0]).wait()
        pltpu.make_async_copy(
            o_bufs[1], out_hbm.at[pl.ds(last + R * G * V, R * G * V)],
            sems[1]).wait()

    return k(x, w)


def kernel(x, w, ind):
    B, dim_x = x.shape
    V = w.shape[1]
    G = dim_x // 2
    return _sc_call(x, w).reshape(B, G, V)
